# baseline (speedup 1.0000x reference)
import functools

import jax
import jax.numpy as jnp
from jax import lax
from jax.experimental import pallas as pl
from jax.experimental.pallas import tpu as pltpu
from jax.experimental.pallas import tpu_sc as plsc

B = 32
D = 512
N = 1024
D_OUT = 2048
NW = 32
DPW = D_OUT // NW
RC = 16
NV = N // 16


def _mm_body(x_ref, a_ref, o_ref):
    xb = x_ref[0]
    a = a_ref[...]
    o_ref[0] = lax.dot_general(
        a, xb, (((0,), (0,)), ((), ())),
        preferred_element_type=jnp.float32,
    )


def _tc_matmul(input, A):
    TQ = 256
    return pl.pallas_call(
        _mm_body,
        grid=(B, D_OUT // TQ),
        in_specs=[
            pl.BlockSpec((1, D, N), lambda b, t: (b, 0, 0)),
            pl.BlockSpec((D, TQ), lambda b, t: (0, t)),
        ],
        out_specs=pl.BlockSpec((1, TQ, N), lambda b, t: (b, t, 0)),
        out_shape=jax.ShapeDtypeStruct((B, D_OUT, N), jnp.float32),
    )(input, A)


def _sig(K, v):
    if K > 1024:
        return False
    return bool(v & (K // 16))


def _sort_row(rows_v, j):
    for v in range(NV):
        s = rows_v[j, pl.ds(16 * v, 16)]
        if _sig(16, v):
            s = -s
        s = plsc.sort_key_val(s, s)[0]
        if _sig(16, v) != _sig(32, v):
            s = -s
        rows_v[j, pl.ds(16 * v, 16)] = s
    for K in (32, 64, 128, 256, 512, 1024):
        d = K // 32
        while d >= 1:
            for a in range(NV):
                if a & d:
                    continue
                x = rows_v[j, pl.ds(16 * a, 16)]
                y = rows_v[j, pl.ds(16 * (a + d), 16)]
                mn = jnp.minimum(x, y)
                mx = jnp.maximum(x, y)
                if d == 1:
                    mn = plsc.sort_key_val(mn, mn)[0]
                    mx = plsc.sort_key_val(mx, mx)[0]
                    if _sig(K, a) != _sig(2 * K, a):
                        mn = -mn
                    if _sig(K, a + 1) != _sig(2 * K, a + 1):
                        mx = -mx
                rows_v[j, pl.ds(16 * a, 16)] = mn
                rows_v[j, pl.ds(16 * (a + d), 16)] = mx
            d //= 2


def _sc_sort_dot(prod, wt):
    mesh = plsc.VectorSubcoreMesh(core_axis_name="c", subcore_axis_name="s")

    @functools.partial(
        pl.kernel,
        mesh=mesh,
        compiler_params=pltpu.CompilerParams(needs_layout_passes=False),
        out_type=jax.ShapeDtypeStruct((B, D_OUT), jnp.float32),
        scratch_types=[
            pltpu.VMEM((DPW, N), jnp.float32),
            pltpu.VMEM((RC, N), jnp.float32),
            pltpu.VMEM((B, DPW), jnp.float32),
        ],
    )
    def k(prod_hbm, wt_hbm, out_hbm, wt_v, rows_v, out_v):
        wid = lax.axis_index("s") * 2 + lax.axis_index("c")
        d0 = wid * DPW
        pltpu.sync_copy(wt_hbm.at[pl.ds(d0, DPW)], wt_v)

        lane = lax.broadcasted_iota(jnp.int32, (16,), 0)

        def chunk_body(m, carry):
            b = m // (DPW // RC)
            sc = m % (DPW // RC)
            pltpu.sync_copy(prod_hbm.at[b, pl.ds(d0 + sc * RC, RC)], rows_v)

            def row_body(j, curr):
                _sort_row(rows_v, j)
                acc = jnp.zeros((16,), jnp.float32)
                for v in range(NV):
                    acc = acc + (rows_v[j, pl.ds(16 * v, 16)]
                                 * wt_v[sc * RC + j, pl.ds(16 * v, 16)])
                tot = jnp.sum(acc)
                return jnp.where(lane == j, tot, curr)

            curr = plsc.parallel_loop(
                0, RC, carry=jnp.zeros((16,), jnp.float32))(row_body)
            out_v[b, pl.ds(sc * RC, 16)] = curr
            return carry

        lax.fori_loop(0, B * (DPW // RC), chunk_body, 0)

        def out_body(b, carry):
            pltpu.sync_copy(out_v.at[b], out_hbm.at[b, pl.ds(d0, DPW)])
            return carry

        lax.fori_loop(0, B, out_body, 0)

    return k(prod, wt)


@jax.jit
def kernel(input, A, w):
    prod = _tc_matmul(input, A)
    wt = jnp.transpose(w[0], (1, 0))
    return _sc_sort_dot(prod, wt)


# speedup vs baseline: 5.8906x; 5.8906x over previous
import jax
import jax.numpy as jnp
from jax.experimental import pallas as pl
from jax.experimental.pallas import tpu as pltpu

N = 1024


def _swap_halves(s, dp):
    R, L = s.shape
    s3 = s.reshape(R // (2 * dp), 2 * dp, L)
    p3 = jnp.concatenate([s3[:, dp:], s3[:, :dp]], axis=1)
    return p3.reshape(R, L)


def _cex_small(s, dp):
    partner = _swap_halves(s, dp)
    mn = jnp.minimum(s, partner)
    mx = jnp.maximum(s, partner)
    ii = jax.lax.broadcasted_iota(jnp.int32, (s.shape[0], 1), 0)
    return jnp.where((ii & dp) == 0, mn, mx)


def _cex_big(s, dp):
    R, L = s.shape
    m = R // (2 * dp)
    s4 = s.reshape(m, 2, dp, L)
    a = s4[:, 0:1]
    b = s4[:, 1:2]
    mn = jnp.minimum(a, b)
    mx = jnp.maximum(a, b)
    return jnp.concatenate([mn, mx], axis=1).reshape(R, L)


def _cex(s, dp):
    return _cex_small(s, dp) if dp < 8 else _cex_big(s, dp)


def _first_levels_pass(s_ref):
    for r0 in range(0, 64, 8):
        q = [s_ref[pl.ds(r0 + 64 * i, 8), :] for i in range(16)]

        def flip(pred):
            for i in range(16):
                if pred(i):
                    q[i] = -q[i]

        def cex_slices(dp):
            dd = dp // 64
            for i in range(16):
                if i & dd:
                    continue
                a, b = q[i], q[i + dd]
                q[i] = jnp.minimum(a, b)
                q[i + dd] = jnp.maximum(a, b)

        flip(lambda i: i & 4)
        cex_slices(512)
        flip(lambda i: bool(i & 4) != bool(i & 2))
        cex_slices(256)
        cex_slices(512)
        flip(lambda i: bool(i & 2) != bool(i & 1))
        cex_slices(128)
        cex_slices(256)
        cex_slices(512)
        flip(lambda i: bool(i & 1) != bool(r0 & 32))
        cex_slices(64)
        cex_slices(128)
        cex_slices(256)
        cex_slices(512)
        for i in range(16):
            s_ref[pl.ds(r0 + 64 * i, 8), :] = q[i]


def _chunk_pass(s_ref, K):
    b_hi = 1024 // K
    b_lo = 512 // K
    for c in range(N // 64):
        r0 = c * 64
        s = s_ref[pl.ds(r0, 64), :]
        ii = r0 + jax.lax.broadcasted_iota(jnp.int32, (64, 1), 0)
        m = (ii & b_hi) != 0
        if b_lo:
            m = m != ((ii & b_lo) != 0)
        s = jnp.where(m, -s, s)
        dp = 1024 // K
        while dp <= 32:
            s = _cex(s, dp)
            dp *= 2
        s_ref[pl.ds(r0, 64), :] = s


def _strided_pass(s_ref):
    for r0 in range(0, 64, 8):
        q = [s_ref[pl.ds(r0 + 64 * i, 8), :] for i in range(16)]
        for dd in (1, 2, 4, 8):
            for i in range(16):
                if i & dd:
                    continue
                a, b = q[i], q[i + dd]
                q[i] = jnp.minimum(a, b)
                q[i + dd] = jnp.maximum(a, b)
        for i in range(16):
            s_ref[pl.ds(r0 + 64 * i, 8), :] = q[i]


def _bitonic_sort_ref(s_ref):
    _first_levels_pass(s_ref)
    for K in (32, 64, 128, 256, 512, 1024):
        _chunk_pass(s_ref, K)
        _strided_pass(s_ref)


def _body(x_ref, a_ref, w_ref, o_ref, s_ref):
    t = pl.program_id(1)
    tile = a_ref.shape[1]
    xb = x_ref[0]
    a = a_ref[...]
    s_ref[...] = jax.lax.dot_general(
        xb, a, (((0,), (0,)), ((), ())),
        preferred_element_type=jnp.float32,
    )
    _bitonic_sort_ref(s_ref)
    wb = w_ref[0]
    o_ref[0, 0, pl.ds(t * tile, tile)] = jnp.sum(s_ref[...] * wb, axis=0)


def _bitrev_perm(n):
    bits = n.bit_length() - 1
    return [int(format(i, f"0{bits}b")[::-1], 2) for i in range(n)]


@jax.jit
def kernel(input, A, w):
    B, D, n = input.shape
    D_OUT = A.shape[1]
    T = 256
    grid = (B, D_OUT // T)
    wp = jnp.take(w, jnp.array(_bitrev_perm(n), dtype=jnp.int32), axis=1)
    return pl.pallas_call(
        _body,
        grid=grid,
        in_specs=[
            pl.BlockSpec((1, D, n), lambda b, t: (b, 0, 0)),
            pl.BlockSpec((D, T), lambda b, t: (0, t)),
            pl.BlockSpec((1, n, T), lambda b, t: (0, 0, t)),
        ],
        out_specs=pl.BlockSpec((1, 1, D_OUT), lambda b, t: (b, 0, 0)),
        out_shape=jax.ShapeDtypeStruct((B, 1, D_OUT), jnp.float32),
        scratch_shapes=[pltpu.VMEM((n, T), jnp.float32)],
    )(input, A, wp)[:, 0, :]


# speedup vs baseline: 6.4541x; 1.0957x over previous
import functools

import jax
import jax.numpy as jnp
from jax import lax
from jax.experimental import pallas as pl
from jax.experimental.pallas import tpu as pltpu
from jax.experimental.pallas import tpu_sc as plsc

N = 1024


def _swap_halves(s, dp):
    R, L = s.shape
    s3 = s.reshape(R // (2 * dp), 2 * dp, L)
    p3 = jnp.concatenate([s3[:, dp:], s3[:, :dp]], axis=1)
    return p3.reshape(R, L)


def _cex_small(s, dp):
    partner = _swap_halves(s, dp)
    mn = jnp.minimum(s, partner)
    mx = jnp.maximum(s, partner)
    ii = jax.lax.broadcasted_iota(jnp.int32, (s.shape[0], 1), 0)
    return jnp.where((ii & dp) == 0, mn, mx)


def _cex_big(s, dp):
    R, L = s.shape
    m = R // (2 * dp)
    s4 = s.reshape(m, 2, dp, L)
    a = s4[:, 0:1]
    b = s4[:, 1:2]
    mn = jnp.minimum(a, b)
    mx = jnp.maximum(a, b)
    return jnp.concatenate([mn, mx], axis=1).reshape(R, L)


def _cex(s, dp):
    return _cex_small(s, dp) if dp < 8 else _cex_big(s, dp)


def _first_levels_pass(s_ref):
    for r0 in range(0, 64, 8):
        q = [s_ref[pl.ds(r0 + 64 * i, 8), :] for i in range(16)]

        def flip(pred):
            for i in range(16):
                if pred(i):
                    q[i] = -q[i]

        def cex_slices(dp):
            dd = dp // 64
            for i in range(16):
                if i & dd:
                    continue
                a, b = q[i], q[i + dd]
                q[i] = jnp.minimum(a, b)
                q[i + dd] = jnp.maximum(a, b)

        flip(lambda i: i & 4)
        cex_slices(512)
        flip(lambda i: bool(i & 4) != bool(i & 2))
        cex_slices(256)
        cex_slices(512)
        flip(lambda i: bool(i & 2) != bool(i & 1))
        cex_slices(128)
        cex_slices(256)
        cex_slices(512)
        flip(lambda i: bool(i & 1) != bool(r0 & 32))
        cex_slices(64)
        cex_slices(128)
        cex_slices(256)
        cex_slices(512)
        for i in range(16):
            s_ref[pl.ds(r0 + 64 * i, 8), :] = q[i]


def _chunk_pass(s_ref, K):
    b_hi = 1024 // K
    b_lo = 512 // K
    for c in range(N // 64):
        r0 = c * 64
        s = s_ref[pl.ds(r0, 64), :]
        ii = r0 + jax.lax.broadcasted_iota(jnp.int32, (64, 1), 0)
        m = (ii & b_hi) != 0
        if b_lo:
            m = m != ((ii & b_lo) != 0)
        s = jnp.where(m, -s, s)
        dp = 1024 // K
        while dp <= 32:
            s = _cex(s, dp)
            dp *= 2
        s_ref[pl.ds(r0, 64), :] = s


def _strided_pass(s_ref):
    for r0 in range(0, 64, 8):
        q = [s_ref[pl.ds(r0 + 64 * i, 8), :] for i in range(16)]
        for dd in (1, 2, 4, 8):
            for i in range(16):
                if i & dd:
                    continue
                a, b = q[i], q[i + dd]
                q[i] = jnp.minimum(a, b)
                q[i + dd] = jnp.maximum(a, b)
        for i in range(16):
            s_ref[pl.ds(r0 + 64 * i, 8), :] = q[i]


def _bitonic_sort_ref(s_ref):
    _first_levels_pass(s_ref)
    for K in (32, 64, 128, 256, 512, 1024):
        _chunk_pass(s_ref, K)
        _strided_pass(s_ref)


def _body(x_ref, a_ref, w_ref, o_ref, s_ref):
    t = pl.program_id(1)
    tile = a_ref.shape[1]
    xb = x_ref[0]
    a = a_ref[...]
    s_ref[...] = jax.lax.dot_general(
        xb, a, (((0,), (0,)), ((), ())),
        preferred_element_type=jnp.float32,
    )
    _bitonic_sort_ref(s_ref)
    wb = w_ref[0]
    o_ref[0, 0, pl.ds(t * tile, tile)] = jnp.sum(s_ref[...] * wb, axis=0)


def _bitrev_perm(n):
    bits = n.bit_length() - 1
    return [int(format(i, f"0{bits}b")[::-1], 2) for i in range(n)]


B = 32
D = 512
D_OUT = 2048
BS = 4
NW = 32
DPW = D_OUT // NW
RC = 16
NV = N // 16


def _mm_body(x_ref, a_ref, o_ref):
    xb = x_ref[0]
    a = a_ref[...]
    o_ref[0] = jax.lax.dot_general(
        a, xb, (((0,), (0,)), ((), ())),
        preferred_element_type=jnp.float32,
    )


def _tc_matmul(input, A):
    return pl.pallas_call(
        _mm_body,
        grid=(BS, 4),
        in_specs=[
            pl.BlockSpec((1, D, N), lambda b, t: (b, 0, 0)),
            pl.BlockSpec((D, D_OUT // 4), lambda b, t: (0, t)),
        ],
        out_specs=pl.BlockSpec((1, D_OUT // 4, N), lambda b, t: (b, t, 0)),
        out_shape=jax.ShapeDtypeStruct((BS, D_OUT, N), jnp.float32),
    )(input, A)


def _sig(K, v):
    if K > 1024:
        return False
    return bool(v & (K // 16))


def _sort_row(rows_v, j):
    for v in range(NV):
        s = rows_v[j, pl.ds(16 * v, 16)]
        if _sig(16, v):
            s = -s
        s = plsc.sort_key_val(s, s)[0]
        if _sig(16, v) != _sig(32, v):
            s = -s
        rows_v[j, pl.ds(16 * v, 16)] = s
    for K in (32, 64, 128, 256, 512, 1024):
        d = K // 32
        while d >= 1:
            for a in range(NV):
                if a & d:
                    continue
                x = rows_v[j, pl.ds(16 * a, 16)]
                y = rows_v[j, pl.ds(16 * (a + d), 16)]
                mn = jnp.minimum(x, y)
                mx = jnp.maximum(x, y)
                if d == 1:
                    mn = plsc.sort_key_val(mn, mn)[0]
                    mx = plsc.sort_key_val(mx, mx)[0]
                    if _sig(K, a) != _sig(2 * K, a):
                        mn = -mn
                    if _sig(K, a + 1) != _sig(2 * K, a + 1):
                        mx = -mx
                rows_v[j, pl.ds(16 * a, 16)] = mn
                rows_v[j, pl.ds(16 * (a + d), 16)] = mx
            d //= 2


def _sc_sort_dot(prod, wt):
    mesh = plsc.VectorSubcoreMesh(core_axis_name="c", subcore_axis_name="s")

    @functools.partial(
        pl.kernel,
        mesh=mesh,
        compiler_params=pltpu.CompilerParams(needs_layout_passes=False),
        out_type=jax.ShapeDtypeStruct((BS, D_OUT), jnp.float32),
        scratch_types=[
            pltpu.VMEM((DPW, N), jnp.float32),
            pltpu.VMEM((RC, N), jnp.float32),
            pltpu.VMEM((BS, DPW), jnp.float32),
        ],
    )
    def k(prod_hbm, wt_hbm, out_hbm, wt_v, rows_v, out_v):
        wid = lax.axis_index("s") * 2 + lax.axis_index("c")
        d0 = wid * DPW
        pltpu.sync_copy(wt_hbm.at[pl.ds(d0, DPW)], wt_v)

        lane = lax.broadcasted_iota(jnp.int32, (16,), 0)

        def chunk_body(m, carry):
            b = m // (DPW // RC)
            sc = m % (DPW // RC)
            pltpu.sync_copy(prod_hbm.at[b, pl.ds(d0 + sc * RC, RC)], rows_v)

            def row_body(j, curr):
                _sort_row(rows_v, j)
                acc = jnp.zeros((16,), jnp.float32)
                for v in range(NV):
                    acc = acc + (rows_v[j, pl.ds(16 * v, 16)]
                                 * wt_v[sc * RC + j, pl.ds(16 * v, 16)])
                tot = jnp.sum(acc)
                return jnp.where(lane == j, tot, curr)

            curr = plsc.parallel_loop(
                0, RC, carry=jnp.zeros((16,), jnp.float32))(row_body)
            out_v[b, pl.ds(sc * RC, 16)] = curr
            return carry

        lax.fori_loop(0, BS * (DPW // RC), chunk_body, 0)

        def out_body(b, carry):
            pltpu.sync_copy(out_v.at[b], out_hbm.at[b, pl.ds(d0, DPW)])
            return carry

        lax.fori_loop(0, BS, out_body, 0)

    return k(prod, wt)


def _tc_sort_part(input, A, wp):
    T = 256
    grid = (B - BS, D_OUT // T)
    return pl.pallas_call(
        _body,
        grid=grid,
        in_specs=[
            pl.BlockSpec((1, D, N), lambda b, t: (b + BS, 0, 0)),
            pl.BlockSpec((D, T), lambda b, t: (0, t)),
            pl.BlockSpec((1, N, T), lambda b, t: (0, 0, t)),
        ],
        out_specs=pl.BlockSpec((1, 1, D_OUT), lambda b, t: (b, 0, 0)),
        out_shape=jax.ShapeDtypeStruct((B - BS, 1, D_OUT), jnp.float32),
        scratch_shapes=[pltpu.VMEM((N, T), jnp.float32)],
    )(input, A, wp)[:, 0, :]


@jax.jit
def kernel(input, A, w):
    wt = jnp.transpose(w[0], (1, 0))
    perm = jnp.array(_bitrev_perm(N), dtype=jnp.int32)
    wp = jnp.take(w, perm, axis=1)
    prod_s = _tc_matmul(input, A)
    out_s = _sc_sort_dot(prod_s, wt)
    out_t = _tc_sort_part(input, A, wp)
    return jnp.concatenate([out_s, out_t], axis=0)
